# baseline (speedup 1.0000x reference)
import jax
import jax.numpy as jnp
from jax import lax
from jax.experimental import pallas as pl
from jax.experimental.pallas import tpu as pltpu
from jax.experimental.pallas import tpu_sc as plsc

_VOCAB = 100000
_DIM = 768
_B = 4
_S = 8192
_NC = 2
_NS = 16
_NW = _NC * _NS
_BPW = (_B * _S) // _NW
_C = 32
_NCHUNK = _BPW // _C
_PAIRS = _DIM // 32
_DEPTH = 4


def _embed3_body(idx_hbm, w0, w1, w2, o0, o1, o2, d0, d1, d2,
                 idx_v, rows0, rows1, rows2, rows3, out0, out1,
                 gsem0, gsem1, gsem2, gsem3, osem0, osem1):
  wid = lax.axis_index("s") * _NC + lax.axis_index("c")
  pltpu.sync_copy(idx_hbm.at[wid], idx_v)
  evens = lax.iota(jnp.int32, 16) * 2
  odds = evens + 1
  rows = (rows0, rows1, rows2, rows3)
  outs = (out0, out1)
  gsems = (gsem0, gsem1, gsem2, gsem3)
  osems = (osem0, osem1)

  def gather(w, cur, b):
    return pltpu.make_async_copy(
        w.at[idx_v.at[pl.ds(cur * _C, _C)]], rows[b], gsems[b])

  def convert(b, ob):
    @plsc.parallel_loop(0, _C)
    def _row(r):
      rvec = jnp.full((16,), r, jnp.int32)
      for j in range(_PAIRS):
        e = plsc.load_gather(rows[b], [rvec, evens + (j * 32)])
        o_ = plsc.load_gather(rows[b], [rvec, odds + (j * 32)])
        outs[ob][r, pl.ds(j * 32, 32)] = plsc.pack(
            e, o_, format=plsc.PackFormat.INTERLEAVED)

  row0 = wid * _BPW
  for t, (w, o, d) in enumerate(((w0, o0, d0), (w1, o1, d1), (w2, o2, d2))):
    for p in range(_DEPTH - 1):
      gather(w, p, p).start()

    @pl.loop(0, _NCHUNK, step=_DEPTH)
    def _chunk4(ci):
      for b in range(_DEPTH):
        cur = ci + b
        nxt = (b + _DEPTH - 1) % _DEPTH
        ob = b % 2

        @pl.when(cur + _DEPTH - 1 < _NCHUNK)
        def _():
          gather(w, cur + _DEPTH - 1, nxt).start()

        gather(w, cur, b).wait()
        def scatter_wait():
          pltpu.make_async_copy(outs[ob], o.at[pl.ds(0, _C)], osems[ob]).wait()
          pltpu.make_async_copy(outs[ob], o.at[pl.ds(0, _C)], osems[ob]).wait()
        if t > 0:
          scatter_wait()
        else:
          pl.when(cur >= 2)(scatter_wait)
        convert(b, ob)
        pltpu.async_copy(outs[ob], o.at[pl.ds(row0 + cur * _C, _C)], osems[ob])
        pltpu.async_copy(outs[ob], d.at[pl.ds(row0 + cur * _C, _C)], osems[ob])

  for ob in range(2):
    pltpu.make_async_copy(outs[ob], o2.at[pl.ds(0, _C)], osems[ob]).wait()
    pltpu.make_async_copy(outs[ob], o2.at[pl.ds(0, _C)], osems[ob]).wait()


@jax.jit
def kernel(inputs, W0, W1, W2):
  idx = inputs.reshape(_NW, _BPW).astype(jnp.int32)
  mesh = plsc.VectorSubcoreMesh(core_axis_name="c", subcore_axis_name="s")
  out_t = jax.ShapeDtypeStruct((_B * _S, _DIM), jnp.bfloat16)
  f = pl.kernel(
      _embed3_body,
      out_type=(out_t, out_t, out_t, out_t, out_t, out_t),
      mesh=mesh,
      compiler_params=pltpu.CompilerParams(needs_layout_passes=False),
      scratch_types=[
          pltpu.VMEM((_BPW,), jnp.int32),
          pltpu.VMEM((_C, _DIM), jnp.float32),
          pltpu.VMEM((_C, _DIM), jnp.float32),
          pltpu.VMEM((_C, _DIM), jnp.float32),
          pltpu.VMEM((_C, _DIM), jnp.float32),
          pltpu.VMEM((_C, _DIM), jnp.bfloat16),
          pltpu.VMEM((_C, _DIM), jnp.bfloat16),
          pltpu.SemaphoreType.DMA,
          pltpu.SemaphoreType.DMA,
          pltpu.SemaphoreType.DMA,
          pltpu.SemaphoreType.DMA,
          pltpu.SemaphoreType.DMA,
          pltpu.SemaphoreType.DMA,
      ],
  )
  o0, o1, o2, d0, d1, d2 = f(idx, W0, W1, W2)
  sh = (_B, _S, _DIM)
  return (o0.reshape(sh), o1.reshape(sh), o2.reshape(sh),
          d0.reshape(sh), d1.reshape(sh), d2.reshape(sh))


# speedup vs baseline: 1.0357x; 1.0357x over previous
import jax
import jax.numpy as jnp
from jax import lax
from jax.experimental import pallas as pl
from jax.experimental.pallas import tpu as pltpu
from jax.experimental.pallas import tpu_sc as plsc

_VOCAB = 100000
_DIM = 768
_B = 4
_S = 8192
_NC = 2
_NS = 16
_NW = _NC * _NS
_BPW = (_B * _S) // _NW
_C = 32
_NCHUNK = _BPW // _C
_PAIRS = _DIM // 32


def _embed3_body(idx_hbm, w0, w1, w2, o0, o1, o2, d0, d1, d2,
                 idx_v, rows0, rows1, out0, out1,
                 gsem0, gsem1, osem0, osem1):
  wid = lax.axis_index("s") * _NC + lax.axis_index("c")
  pltpu.sync_copy(idx_hbm.at[wid], idx_v)
  evens = lax.iota(jnp.int32, 16) * 2
  odds = evens + 1
  rows = (rows0, rows1)
  outs = (out0, out1)
  gsems = (gsem0, gsem1)
  osems = (osem0, osem1)

  def gather(w, cur, b):
    return pltpu.make_async_copy(
        w.at[idx_v.at[pl.ds(cur * _C, _C)]], rows[b], gsems[b])

  def convert(b):
    @plsc.parallel_loop(0, _C)
    def _row(r):
      rvec = jnp.full((16,), r, jnp.int32)
      for j in range(_PAIRS):
        e = plsc.load_gather(rows[b], [rvec, evens + (j * 32)])
        o_ = plsc.load_gather(rows[b], [rvec, odds + (j * 32)])
        outs[b][r, pl.ds(j * 32, 32)] = plsc.pack(
            e, o_, format=plsc.PackFormat.INTERLEAVED)

  row0 = wid * _BPW
  tables = ((w0, o0, d0), (w1, o1, d1), (w2, o2, d2))
  for t, (w, o, d) in enumerate(tables):
    if t == 0:
      gather(w, 0, 0).start()
    w_next = tables[t + 1][0] if t < 2 else None

    @pl.loop(0, _NCHUNK, step=2)
    def _chunk2(ci):
      for b in range(2):
        cur = ci + b
        nxt = 1 - b

        @pl.when(cur + 1 < _NCHUNK)
        def _():
          gather(w, cur + 1, nxt).start()

        if w_next is not None and b == 1:
          @pl.when(cur + 1 == _NCHUNK)
          def _():
            gather(w_next, 0, nxt).start()

        gather(w, cur, b).wait()
        def scatter_wait():
          pltpu.make_async_copy(outs[b], o.at[pl.ds(0, _C)], osems[b]).wait()
          pltpu.make_async_copy(outs[b], o.at[pl.ds(0, _C)], osems[b]).wait()
        if t > 0:
          scatter_wait()
        else:
          pl.when(cur >= 2)(scatter_wait)
        convert(b)
        pltpu.async_copy(outs[b], o.at[pl.ds(row0 + cur * _C, _C)], osems[b])
        pltpu.async_copy(outs[b], d.at[pl.ds(row0 + cur * _C, _C)], osems[b])

  for b in range(2):
    pltpu.make_async_copy(outs[b], o2.at[pl.ds(0, _C)], osems[b]).wait()
    pltpu.make_async_copy(outs[b], o2.at[pl.ds(0, _C)], osems[b]).wait()


@jax.jit
def kernel(inputs, W0, W1, W2):
  idx = inputs.reshape(_NW, _BPW).astype(jnp.int32)
  mesh = plsc.VectorSubcoreMesh(core_axis_name="c", subcore_axis_name="s")
  out_t = jax.ShapeDtypeStruct((_B * _S, _DIM), jnp.bfloat16)
  f = pl.kernel(
      _embed3_body,
      out_type=(out_t, out_t, out_t, out_t, out_t, out_t),
      mesh=mesh,
      compiler_params=pltpu.CompilerParams(needs_layout_passes=False),
      scratch_types=[
          pltpu.VMEM((_BPW,), jnp.int32),
          pltpu.VMEM((_C, _DIM), jnp.float32),
          pltpu.VMEM((_C, _DIM), jnp.float32),
          pltpu.VMEM((_C, _DIM), jnp.bfloat16),
          pltpu.VMEM((_C, _DIM), jnp.bfloat16),
          pltpu.SemaphoreType.DMA,
          pltpu.SemaphoreType.DMA,
          pltpu.SemaphoreType.DMA,
          pltpu.SemaphoreType.DMA,
      ],
  )
  o0, o1, o2, d0, d1, d2 = f(idx, W0, W1, W2)
  sh = (_B, _S, _DIM)
  return (o0.reshape(sh), o1.reshape(sh), o2.reshape(sh),
          d0.reshape(sh), d1.reshape(sh), d2.reshape(sh))
